# baseline (speedup 1.0000x reference)
import jax
import jax.numpy as jnp
from jax.experimental import pallas as pl
from jax.experimental.pallas import tpu as pltpu

_NCHUNK = 4
_RC = 128 // _NCHUNK


def _dotT(a, b):
    return jax.lax.dot_general(
        a, b, (((1,), (1,)), ((), ())), preferred_element_type=jnp.float32
    )


def _gate_body(z_hbm, tf_hbm, tr_hbm, w1_hbm, b1_hbm, a_hbm,
               lnw_hbm, lnb_hbm, w2_hbm, b2_hbm, out_ref,
               z_v, tf_v, tr_v, b1_v, a_v, lnw_v, lnb_v, w2_v, b2_v,
               c0, c1, c2, c3, sem_z, sem_small, s0, s1, s2, s3):
    chunk_bufs = (c0, c1, c2, c3)
    chunk_sems = (s0, s1, s2, s3)

    cp_z = pltpu.make_async_copy(z_hbm, z_v, sem_z)
    cp_z.start()
    chunk_cps = [
        pltpu.make_async_copy(
            w1_hbm.at[pl.ds(i * _RC, _RC), :], chunk_bufs[i], chunk_sems[i])
        for i in range(_NCHUNK)
    ]
    for cp in chunk_cps:
        cp.start()
    small = [
        pltpu.make_async_copy(tf_hbm, tf_v, sem_small),
        pltpu.make_async_copy(tr_hbm, tr_v, sem_small),
        pltpu.make_async_copy(b1_hbm, b1_v, sem_small),
        pltpu.make_async_copy(a_hbm, a_v, sem_small),
        pltpu.make_async_copy(lnw_hbm, lnw_v, sem_small),
        pltpu.make_async_copy(lnb_hbm, lnb_v, sem_small),
        pltpu.make_async_copy(w2_hbm, w2_v, sem_small),
        pltpu.make_async_copy(b2_hbm, b2_v, sem_small),
    ]
    for cp in small:
        cp.start()

    cp_z.wait()
    segs = []
    for i in range(_NCHUNK):
        chunk_cps[i].wait()
        buf = chunk_bufs[i]
        segs.append(_dotT(z_v[...], buf[:, 0:4096]))
    h = jnp.concatenate(segs, axis=1)

    for cp in small:
        cp.wait()
    tails = [jnp.concatenate([b[:, 4096:4097] for b in chunk_bufs], axis=0),
             jnp.concatenate([b[:, 4097:4098] for b in chunk_bufs], axis=0)]
    h = h + _dotT(tf_v[...], tails[0])
    h = h + _dotT(tr_v[...], tails[1])
    h = h + b1_v[...]
    h = jnp.maximum(h, 0.0) + a_v[...] * jnp.minimum(h, 0.0)
    mu = jnp.mean(h, axis=1, keepdims=True)
    d = h - mu
    var = jnp.mean(d * d, axis=1, keepdims=True)
    hn = d * jax.lax.rsqrt(var + 1e-5) * lnw_v[...] + lnb_v[...]
    logits = _dotT(hn, w2_v[...]) + b2_v[...]
    m = jnp.max(logits, axis=1, keepdims=True)
    e = jnp.exp(logits - m)
    s = jnp.sum(e, axis=1, keepdims=True)
    out_ref[...] = e / s


@jax.jit
def _gate(z, tf, tr, W1, b1, a, lnw, lnb, W2, b2):
    hbm = pl.BlockSpec(memory_space=pltpu.MemorySpace.HBM)
    return pl.pallas_call(
        _gate_body,
        out_shape=jax.ShapeDtypeStruct((1, 64), jnp.float32),
        in_specs=[hbm] * 10,
        out_specs=pl.BlockSpec(memory_space=pltpu.MemorySpace.VMEM),
        scratch_shapes=[
            pltpu.VMEM((1, 4096), jnp.float32),
            pltpu.VMEM((1, 1), jnp.float32),
            pltpu.VMEM((1, 1), jnp.float32),
            pltpu.VMEM((1, 128), jnp.float32),
            pltpu.VMEM((1, 1), jnp.float32),
            pltpu.VMEM((1, 128), jnp.float32),
            pltpu.VMEM((1, 128), jnp.float32),
            pltpu.VMEM((64, 128), jnp.float32),
            pltpu.VMEM((1, 64), jnp.float32),
            pltpu.VMEM((_RC, 4098), jnp.float32),
            pltpu.VMEM((_RC, 4098), jnp.float32),
            pltpu.VMEM((_RC, 4098), jnp.float32),
            pltpu.VMEM((_RC, 4098), jnp.float32),
            pltpu.SemaphoreType.DMA,
            pltpu.SemaphoreType.DMA,
            pltpu.SemaphoreType.DMA,
            pltpu.SemaphoreType.DMA,
            pltpu.SemaphoreType.DMA,
            pltpu.SemaphoreType.DMA,
        ],
    )(z, tf, tr, W1, b1, a, lnw, lnb, W2, b2)


def kernel(z_refined, trust_form, trust_role, W1, b1, prelu_a, ln_w, ln_b, W2, b2):
    return _gate(
        z_refined,
        trust_form.reshape(1, 1),
        trust_role.reshape(1, 1),
        W1,
        b1.reshape(1, 128),
        prelu_a.reshape(1, 1),
        ln_w.reshape(1, 128),
        ln_b.reshape(1, 128),
        W2,
        b2.reshape(1, 64),
    )
